# baseline (speedup 1.0000x reference)
import functools

import jax
import jax.numpy as jnp
import numpy as np
from jax import lax
from jax.experimental import pallas as pl
from jax.experimental.pallas import tpu as pltpu
from jax.experimental.pallas import tpu_sc as plsc

_B = 32
_V = 1000000
_EPS = 1e-10


def _threefry2x32_np(k0, k1, x0, x1):
    ks0 = np.uint32(k0)
    ks1 = np.uint32(k1)
    ks2 = np.uint32(ks0 ^ ks1 ^ np.uint32(0x1BD11BDA))
    ks = [ks0, ks1, ks2]
    rotations = [[13, 15, 26, 6], [17, 29, 16, 24]]
    x0 = x0 + ks0
    x1 = x1 + ks1

    def rotl(x, r):
        return (x << np.uint32(r)) | (x >> np.uint32(32 - r))

    for i in range(5):
        for r in rotations[i % 2]:
            x0 = x0 + x1
            x1 = rotl(x1, r)
            x1 = x1 ^ x0
        x0 = x0 + ks[(i + 1) % 3]
        x1 = x1 + ks[(i + 2) % 3] + np.uint32(i + 1)
    return x0, x1


def _log_noise_table():
    n = _B * _V
    x1 = np.arange(n, dtype=np.uint32)
    x0 = np.zeros(n, dtype=np.uint32)
    o0, o1 = _threefry2x32_np(0, 42, x0, x1)
    bits = o0 ^ o1
    del x0, x1, o0, o1
    u = ((bits >> np.uint32(9)) | np.uint32(0x3F800000)).view(np.float32)
    u = u - np.float32(1.0)
    del bits
    e64 = -np.log1p(-u.astype(np.float64))
    del u
    return np.log(e64 + _EPS).astype(np.float32).reshape(_B, _V)


_LE_NP = _log_noise_table()

_LANES = 16
_ROWS = 8
_NSTRIPE = 16
_WC = 1024
_NCHUNK = 61
_STRIPE = _WC * _NCHUNK
_EXTRA_COL = _NSTRIPE * _STRIPE
_EXTRA_W = 512
_TAILIN_COL = _EXTRA_COL + _EXTRA_W
_TAILIN_W = 128
_UNROLL = 16
_NEG_INF = float("-inf")

_mesh = plsc.VectorSubcoreMesh(core_axis_name="c", subcore_axis_name="s",
                               num_cores=1)


@functools.partial(
    pl.kernel,
    out_type=(jax.ShapeDtypeStruct((_ROWS, _LANES), jnp.int32),
              jax.ShapeDtypeStruct((_NSTRIPE * _LANES,), jnp.float32),
              jax.ShapeDtypeStruct((_NSTRIPE * _LANES,), jnp.int32)),
    mesh=_mesh,
    compiler_params=pltpu.CompilerParams(needs_layout_passes=False),
    scratch_types=[
        pltpu.VMEM((2, _ROWS, _WC), jnp.float32),
        pltpu.VMEM((2, _ROWS, _WC), jnp.float32),
        pltpu.VMEM((_ROWS, _EXTRA_W), jnp.float32),
        pltpu.VMEM((_ROWS, _EXTRA_W), jnp.float32),
        pltpu.VMEM((_ROWS, _TAILIN_W), jnp.float32),
        pltpu.VMEM((_ROWS, _TAILIN_W), jnp.float32),
        pltpu.VMEM((_ROWS, 128), jnp.float32),
        pltpu.VMEM((_LANES,), jnp.float32),
        pltpu.VMEM((_LANES,), jnp.int32),
        pltpu.VMEM((_NSTRIPE * _LANES,), jnp.float32),
        pltpu.VMEM((_NSTRIPE * _LANES,), jnp.int32),
        pltpu.VMEM((_ROWS, _LANES), jnp.int32),
        pltpu.SemaphoreType.DMA,
        pltpu.SemaphoreType.DMA,
        pltpu.SemaphoreType.DMA,
        pltpu.SemaphoreType.DMA,
        pltpu.SemaphoreType.DMA,
    ],
)
def _sc_sample(logits_hbm, le_hbm, ltail_hbm, ntail_hbm, temps_hbm,
               out_hbm, pvals_hbm, pidxs_hbm,
               lbuf, nbuf, lxbuf, nxbuf, ltbuf, ntbuf, tbuf, pvbuf, pibuf,
               mv, mi2, rbuf, sem0, sem1, sem2, sem3, semt):
    t = lax.axis_index("s")
    cbase = t * _STRIPE

    pltpu.sync_copy(temps_hbm.at[pl.ds(0, _ROWS)], tbuf)
    lane = jnp.arange(_LANES, dtype=jnp.int32)
    tvecs = [tbuf[r, pl.ds(0, _LANES)] for r in range(_ROWS)]

    lsems = (sem0, sem1)
    nsems = (sem2, sem3)

    def _start(c, slot):
        col = pl.multiple_of(cbase + c * _WC, 128)
        pltpu.async_copy(
            logits_hbm.at[pl.ds(0, _ROWS), pl.ds(col, _WC)],
            lbuf.at[slot], lsems[slot])
        pltpu.async_copy(
            le_hbm.at[pl.ds(0, _ROWS), pl.ds(col, _WC)],
            nbuf.at[slot], nsems[slot])

    def _wait(slot):
        pltpu.make_async_copy(
            logits_hbm.at[pl.ds(0, _ROWS), pl.ds(0, _WC)],
            lbuf.at[slot], lsems[slot]).wait()
        pltpu.make_async_copy(
            le_hbm.at[pl.ds(0, _ROWS), pl.ds(0, _WC)],
            nbuf.at[slot], nsems[slot]).wait()

    def _fold(lref, nref, sub, goff, n_iters, carry, unroll=_UNROLL):
        ms, mis = carry
        new_ms, new_mis = [], []
        for r in range(_ROWS):
            tv = tvecs[r]

            def body(i, c2, r=r, tv=tv):
                m, mi_v = c2
                for u in range(unroll):
                    off = i * (unroll * _LANES) + u * _LANES
                    lv = lref[sub + (r, pl.ds(off, _LANES))]
                    nv = nref[sub + (r, pl.ds(off, _LANES))]
                    s = lv - tv * nv
                    gidx = lane + (goff + off)
                    upd = s > m
                    m = jnp.where(upd, s, m)
                    mi_v = jnp.where(upd, gidx, mi_v)
                return m, mi_v

            m, mi_v = lax.fori_loop(0, n_iters, body, (ms[r], mis[r]))
            new_ms.append(m)
            new_mis.append(mi_v)
        return new_ms, new_mis

    def _consume(c, slot, carry):
        _wait(slot)
        carry = _fold(lbuf, nbuf, (slot,), cbase + c * _WC,
                      _WC // (_UNROLL * _LANES), carry)

        @pl.when(c + 2 < _NCHUNK)
        def _():
            _start(c + 2, slot)

        return carry

    _start(0, 0)
    _start(1, 1)

    def chunk_pair(j, carry):
        c0 = 2 * j
        carry = _consume(c0, 0, carry)
        carry = _consume(c0 + 1, 1, carry)
        return carry

    m0 = [jnp.full((_LANES,), _NEG_INF, jnp.float32) for _ in range(_ROWS)]
    mi0 = [jnp.zeros((_LANES,), jnp.int32) for _ in range(_ROWS)]
    carry = lax.fori_loop(0, (_NCHUNK - 1) // 2, chunk_pair, (m0, mi0))
    carry = _consume(_NCHUNK - 1, 0, carry)

    def _grab(src, dst, cols):
        pltpu.async_copy(src.at[pl.ds(0, _ROWS), cols], dst, semt)
        pltpu.make_async_copy(
            src.at[pl.ds(0, _ROWS), cols], dst, semt).wait()

    _grab(logits_hbm, lxbuf, pl.ds(_EXTRA_COL, _EXTRA_W))
    _grab(le_hbm, nxbuf, pl.ds(_EXTRA_COL, _EXTRA_W))
    carry = _fold(lxbuf, nxbuf, (), _EXTRA_COL,
                  _EXTRA_W // (_UNROLL * _LANES), carry)
    pltpu.async_copy(ltail_hbm.at[pl.ds(0, _ROWS)], ltbuf, semt)
    pltpu.make_async_copy(
        ltail_hbm.at[pl.ds(0, _ROWS)], ltbuf, semt).wait()
    pltpu.async_copy(ntail_hbm.at[pl.ds(0, _ROWS)], ntbuf, semt)
    pltpu.make_async_copy(
        ntail_hbm.at[pl.ds(0, _ROWS)], ntbuf, semt).wait()
    ms, mis = _fold(ltbuf, ntbuf, (), _TAILIN_COL, 1, carry,
                    unroll=_TAILIN_W // _LANES)

    pv = jnp.full((_LANES,), _NEG_INF, jnp.float32)
    pi = jnp.zeros((_LANES,), jnp.int32)
    for r in range(_ROWS):
        gm = jnp.max(ms[r])
        best = jnp.min(jnp.where(ms[r] == gm, mis[r], _V))
        pv = jnp.where(lane == r, jnp.full((_LANES,), gm), pv)
        pi = jnp.where(lane == r, jnp.full((_LANES,), best), pi)
    pvbuf[...] = pv
    pibuf[...] = pi
    soff = pl.multiple_of(t * _LANES, 8)
    pltpu.sync_copy(pvbuf, pvals_hbm.at[pl.ds(soff, _LANES)])
    pltpu.sync_copy(pibuf, pidxs_hbm.at[pl.ds(soff, _LANES)])
    plsc.subcore_barrier()

    @pl.when(t == 0)
    def _():
        pltpu.sync_copy(pvals_hbm, mv)
        pltpu.sync_copy(pidxs_hbm, mi2)
        bv = jnp.full((_LANES,), _NEG_INF, jnp.float32)
        bi = jnp.full((_LANES,), _V, jnp.int32)
        for tt in range(_NSTRIPE):
            v = mv[pl.ds(tt * _LANES, _LANES)]
            i = mi2[pl.ds(tt * _LANES, _LANES)]
            gt = v > bv
            eq = v == bv
            bi = jnp.where(gt, i, jnp.where(eq, jnp.minimum(i, bi), bi))
            bv = jnp.where(gt, v, bv)
        for r in range(_ROWS):
            best = jnp.min(jnp.where(lane == r, bi, _V))
            rbuf[r, :] = jnp.full((_LANES,), best, jnp.int32)
        pltpu.sync_copy(rbuf, out_hbm)


_LE_TAIL_NP = np.zeros((_B, _TAILIN_W), np.float32)
_LE_TAIL_NP[:, : _V - _TAILIN_COL] = _LE_NP[:, _TAILIN_COL:]

_B_TC = _B - _ROWS
_RB = 8
_NRB = _B_TC // _RB
_WTC = 131072
_NTC = (_V + _WTC - 1) // _WTC


def _tc_body(lref, nref, tref, oref, mref, miref):
    j = pl.program_id(0)
    i = pl.program_id(1)

    @pl.when(i == 0)
    def _():
        mref[...] = jnp.full((_RB, 1), _NEG_INF, jnp.float32)
        miref[...] = jnp.zeros((_RB, 1), jnp.int32)

    tv = tref[:, :1]
    s = lref[...] - tv * nref[...]
    col = jax.lax.broadcasted_iota(jnp.int32, (_RB, _WTC), 1) + i * _WTC
    s = jnp.where(col < _V, s, _NEG_INF)
    bm = jnp.max(s, axis=1, keepdims=True)
    bi = jnp.min(jnp.where(s == bm, col, _V), axis=1, keepdims=True)
    m = mref[...]
    mi_v = miref[...]
    upd = bm > m
    mref[...] = jnp.where(upd, bm, m)
    miref[...] = jnp.where(upd, bi, mi_v)

    @pl.when(i == _NTC - 1)
    def _():
        oref[...] = miref[...]


_tc_sample = pl.pallas_call(
    _tc_body,
    grid=(_NRB, _NTC),
    in_specs=[
        pl.BlockSpec((_RB, _WTC), lambda j, i: (j + 1, i)),
        pl.BlockSpec((_RB, _WTC), lambda j, i: (j + 1, i)),
        pl.BlockSpec((_RB, 128), lambda j, i: (j + 1, 0)),
    ],
    out_specs=pl.BlockSpec((_RB, 1), lambda j, i: (j, 0)),
    out_shape=jax.ShapeDtypeStruct((_B_TC, 1), jnp.int32),
    scratch_shapes=[
        pltpu.VMEM((_RB, 1), jnp.float32),
        pltpu.VMEM((_RB, 1), jnp.int32),
    ],
    compiler_params=pltpu.CompilerParams(
        dimension_semantics=("arbitrary", "arbitrary")),
)


def kernel(logits, temperatures):
    le = jnp.asarray(_LE_NP)
    ltail = jnp.pad(logits[:, _TAILIN_COL:],
                    ((0, 0), (0, _TAILIN_W - (_V - _TAILIN_COL))),
                    constant_values=_NEG_INF)
    ntail = jnp.asarray(_LE_TAIL_NP)
    temps_b = jnp.broadcast_to(temperatures[:, None], (_B, 128))
    out_sc, _, _ = _sc_sample(logits, le, ltail, ntail, temps_b)
    out_tc = _tc_sample(logits, le, temps_b)
    return jnp.concatenate([out_sc[:, 0], out_tc[:, 0]])


# speedup vs baseline: 1.0890x; 1.0890x over previous
import functools

import jax
import jax.numpy as jnp
import numpy as np
from jax import lax
from jax.experimental import pallas as pl
from jax.experimental.pallas import tpu as pltpu
from jax.experimental.pallas import tpu_sc as plsc

_B = 32
_V = 1000000
_EPS = 1e-10


def _threefry2x32_np(k0, k1, x0, x1):
    ks0 = np.uint32(k0)
    ks1 = np.uint32(k1)
    ks2 = np.uint32(ks0 ^ ks1 ^ np.uint32(0x1BD11BDA))
    ks = [ks0, ks1, ks2]
    rotations = [[13, 15, 26, 6], [17, 29, 16, 24]]
    x0 = x0 + ks0
    x1 = x1 + ks1

    def rotl(x, r):
        return (x << np.uint32(r)) | (x >> np.uint32(32 - r))

    for i in range(5):
        for r in rotations[i % 2]:
            x0 = x0 + x1
            x1 = rotl(x1, r)
            x1 = x1 ^ x0
        x0 = x0 + ks[(i + 1) % 3]
        x1 = x1 + ks[(i + 2) % 3] + np.uint32(i + 1)
    return x0, x1


def _log_noise_table():
    n = _B * _V
    x1 = np.arange(n, dtype=np.uint32)
    x0 = np.zeros(n, dtype=np.uint32)
    o0, o1 = _threefry2x32_np(0, 42, x0, x1)
    bits = o0 ^ o1
    del x0, x1, o0, o1
    u = ((bits >> np.uint32(9)) | np.uint32(0x3F800000)).view(np.float32)
    u = u - np.float32(1.0)
    del bits
    e64 = -np.log1p(-u.astype(np.float64))
    del u
    return np.log(e64 + _EPS).astype(np.float32).reshape(_B, _V)


_LE_NP = _log_noise_table()

_LANES = 16
_ROWS = 8
_NSTRIPE = 16
_WC = 1024
_NCHUNK = 61
_STRIPE = _WC * _NCHUNK
_EXTRA_COL = _NSTRIPE * _STRIPE
_EXTRA_W = 512
_TAILIN_COL = _EXTRA_COL + _EXTRA_W
_TAILIN_W = 128
_UNROLL = 8
_NEG_INF = float("-inf")

_mesh = plsc.VectorSubcoreMesh(core_axis_name="c", subcore_axis_name="s",
                               num_cores=1)


@functools.partial(
    pl.kernel,
    out_type=(jax.ShapeDtypeStruct((_ROWS, _LANES), jnp.int32),
              jax.ShapeDtypeStruct((_NSTRIPE * _LANES,), jnp.float32),
              jax.ShapeDtypeStruct((_NSTRIPE * _LANES,), jnp.int32)),
    mesh=_mesh,
    compiler_params=pltpu.CompilerParams(needs_layout_passes=False),
    scratch_types=[
        pltpu.VMEM((2, _ROWS, _WC), jnp.float32),
        pltpu.VMEM((2, _ROWS, _WC), jnp.float32),
        pltpu.VMEM((_ROWS, _EXTRA_W), jnp.float32),
        pltpu.VMEM((_ROWS, _EXTRA_W), jnp.float32),
        pltpu.VMEM((_ROWS, _TAILIN_W), jnp.float32),
        pltpu.VMEM((_ROWS, _TAILIN_W), jnp.float32),
        pltpu.VMEM((_ROWS, 128), jnp.float32),
        pltpu.VMEM((_LANES,), jnp.float32),
        pltpu.VMEM((_LANES,), jnp.int32),
        pltpu.VMEM((_NSTRIPE * _LANES,), jnp.float32),
        pltpu.VMEM((_NSTRIPE * _LANES,), jnp.int32),
        pltpu.VMEM((_ROWS, _LANES), jnp.int32),
        pltpu.SemaphoreType.DMA,
        pltpu.SemaphoreType.DMA,
        pltpu.SemaphoreType.DMA,
        pltpu.SemaphoreType.DMA,
        pltpu.SemaphoreType.DMA,
    ],
)
def _sc_sample(logits_hbm, le_hbm, ltail_hbm, ntail_hbm, temps_hbm,
               out_hbm, pvals_hbm, pidxs_hbm,
               lbuf, nbuf, lxbuf, nxbuf, ltbuf, ntbuf, tbuf, pvbuf, pibuf,
               mv, mi2, rbuf, sem0, sem1, sem2, sem3, semt):
    t = lax.axis_index("s")
    cbase = t * _STRIPE

    pltpu.sync_copy(temps_hbm.at[pl.ds(0, _ROWS)], tbuf)
    lane = jnp.arange(_LANES, dtype=jnp.int32)
    tvecs = [tbuf[r, pl.ds(0, _LANES)] for r in range(_ROWS)]

    lsems = (sem0, sem1)
    nsems = (sem2, sem3)

    def _start(c, slot):
        col = pl.multiple_of(cbase + c * _WC, 128)
        pltpu.async_copy(
            logits_hbm.at[pl.ds(0, _ROWS), pl.ds(col, _WC)],
            lbuf.at[slot], lsems[slot])
        pltpu.async_copy(
            le_hbm.at[pl.ds(0, _ROWS), pl.ds(col, _WC)],
            nbuf.at[slot], nsems[slot])

    def _wait(slot):
        pltpu.make_async_copy(
            logits_hbm.at[pl.ds(0, _ROWS), pl.ds(0, _WC)],
            lbuf.at[slot], lsems[slot]).wait()
        pltpu.make_async_copy(
            le_hbm.at[pl.ds(0, _ROWS), pl.ds(0, _WC)],
            nbuf.at[slot], nsems[slot]).wait()

    def _fold(lref, nref, sub, goff, n_iters, carry, unroll=_UNROLL):
        ms, mis = carry
        new_ms, new_mis = [], []
        for r in range(_ROWS):
            tv = tvecs[r]

            def body(i, c2, r=r, tv=tv):
                m, mi_v = c2
                for u in range(unroll):
                    off = i * (unroll * _LANES) + u * _LANES
                    lv = lref[sub + (r, pl.ds(off, _LANES))]
                    nv = nref[sub + (r, pl.ds(off, _LANES))]
                    s = lv - tv * nv
                    gidx = lane + (goff + off)
                    upd = s > m
                    m = jnp.where(upd, s, m)
                    mi_v = jnp.where(upd, gidx, mi_v)
                return m, mi_v

            m, mi_v = lax.fori_loop(0, n_iters, body, (ms[r], mis[r]))
            new_ms.append(m)
            new_mis.append(mi_v)
        return new_ms, new_mis

    def _consume(c, slot, carry):
        _wait(slot)
        carry = _fold(lbuf, nbuf, (slot,), cbase + c * _WC,
                      _WC // (_UNROLL * _LANES), carry)

        @pl.when(c + 2 < _NCHUNK)
        def _():
            _start(c + 2, slot)

        return carry

    _start(0, 0)
    _start(1, 1)

    def chunk_pair(j, carry):
        c0 = 2 * j
        carry = _consume(c0, 0, carry)
        carry = _consume(c0 + 1, 1, carry)
        return carry

    m0 = [jnp.full((_LANES,), _NEG_INF, jnp.float32) for _ in range(_ROWS)]
    mi0 = [jnp.zeros((_LANES,), jnp.int32) for _ in range(_ROWS)]
    carry = lax.fori_loop(0, (_NCHUNK - 1) // 2, chunk_pair, (m0, mi0))
    carry = _consume(_NCHUNK - 1, 0, carry)

    def _grab(src, dst, cols):
        pltpu.async_copy(src.at[pl.ds(0, _ROWS), cols], dst, semt)
        pltpu.make_async_copy(
            src.at[pl.ds(0, _ROWS), cols], dst, semt).wait()

    _grab(logits_hbm, lxbuf, pl.ds(_EXTRA_COL, _EXTRA_W))
    _grab(le_hbm, nxbuf, pl.ds(_EXTRA_COL, _EXTRA_W))
    carry = _fold(lxbuf, nxbuf, (), _EXTRA_COL,
                  _EXTRA_W // (_UNROLL * _LANES), carry)
    pltpu.async_copy(ltail_hbm.at[pl.ds(0, _ROWS)], ltbuf, semt)
    pltpu.make_async_copy(
        ltail_hbm.at[pl.ds(0, _ROWS)], ltbuf, semt).wait()
    pltpu.async_copy(ntail_hbm.at[pl.ds(0, _ROWS)], ntbuf, semt)
    pltpu.make_async_copy(
        ntail_hbm.at[pl.ds(0, _ROWS)], ntbuf, semt).wait()
    ms, mis = _fold(ltbuf, ntbuf, (), _TAILIN_COL, 1, carry,
                    unroll=_TAILIN_W // _LANES)

    pv = jnp.full((_LANES,), _NEG_INF, jnp.float32)
    pi = jnp.zeros((_LANES,), jnp.int32)
    for r in range(_ROWS):
        gm = jnp.max(ms[r])
        best = jnp.min(jnp.where(ms[r] == gm, mis[r], _V))
        pv = jnp.where(lane == r, jnp.full((_LANES,), gm), pv)
        pi = jnp.where(lane == r, jnp.full((_LANES,), best), pi)
    pvbuf[...] = pv
    pibuf[...] = pi
    soff = pl.multiple_of(t * _LANES, 8)
    pltpu.sync_copy(pvbuf, pvals_hbm.at[pl.ds(soff, _LANES)])
    pltpu.sync_copy(pibuf, pidxs_hbm.at[pl.ds(soff, _LANES)])
    plsc.subcore_barrier()

    @pl.when(t == 0)
    def _():
        pltpu.sync_copy(pvals_hbm, mv)
        pltpu.sync_copy(pidxs_hbm, mi2)
        bv = jnp.full((_LANES,), _NEG_INF, jnp.float32)
        bi = jnp.full((_LANES,), _V, jnp.int32)
        for tt in range(_NSTRIPE):
            v = mv[pl.ds(tt * _LANES, _LANES)]
            i = mi2[pl.ds(tt * _LANES, _LANES)]
            gt = v > bv
            eq = v == bv
            bi = jnp.where(gt, i, jnp.where(eq, jnp.minimum(i, bi), bi))
            bv = jnp.where(gt, v, bv)
        for r in range(_ROWS):
            best = jnp.min(jnp.where(lane == r, bi, _V))
            rbuf[r, :] = jnp.full((_LANES,), best, jnp.int32)
        pltpu.sync_copy(rbuf, out_hbm)


_LE_TAIL_NP = np.zeros((_B, _TAILIN_W), np.float32)
_LE_TAIL_NP[:, : _V - _TAILIN_COL] = _LE_NP[:, _TAILIN_COL:]

_B_TC = _B - _ROWS
_RB = 8
_NRB = _B_TC // _RB
_WTC = 131072
_NTC = (_V + _WTC - 1) // _WTC


def _tc_body(lref, nref, tref, oref, mref, miref):
    j = pl.program_id(0)
    i = pl.program_id(1)

    @pl.when(i == 0)
    def _():
        mref[...] = jnp.full((_RB, 1), _NEG_INF, jnp.float32)
        miref[...] = jnp.zeros((_RB, 1), jnp.int32)

    tv = tref[:, :1]
    s = lref[...] - tv * nref[...]
    col = jax.lax.broadcasted_iota(jnp.int32, (_RB, _WTC), 1) + i * _WTC
    s = jnp.where(col < _V, s, _NEG_INF)
    bm = jnp.max(s, axis=1, keepdims=True)
    bi = jnp.min(jnp.where(s == bm, col, _V), axis=1, keepdims=True)
    m = mref[...]
    mi_v = miref[...]
    upd = bm > m
    mref[...] = jnp.where(upd, bm, m)
    miref[...] = jnp.where(upd, bi, mi_v)

    @pl.when(i == _NTC - 1)
    def _():
        oref[...] = miref[...]


_tc_sample = pl.pallas_call(
    _tc_body,
    grid=(_NRB, _NTC),
    in_specs=[
        pl.BlockSpec((_RB, _WTC), lambda j, i: (j + 1, i)),
        pl.BlockSpec((_RB, _WTC), lambda j, i: (j + 1, i)),
        pl.BlockSpec((_RB, 128), lambda j, i: (j + 1, 0)),
    ],
    out_specs=pl.BlockSpec((_RB, 1), lambda j, i: (j, 0)),
    out_shape=jax.ShapeDtypeStruct((_B_TC, 1), jnp.int32),
    scratch_shapes=[
        pltpu.VMEM((_RB, 1), jnp.float32),
        pltpu.VMEM((_RB, 1), jnp.int32),
    ],
    compiler_params=pltpu.CompilerParams(
        dimension_semantics=("arbitrary", "arbitrary")),
)


def kernel(logits, temperatures):
    le = jnp.asarray(_LE_NP)
    ltail = jnp.pad(logits[:, _TAILIN_COL:],
                    ((0, 0), (0, _TAILIN_W - (_V - _TAILIN_COL))),
                    constant_values=_NEG_INF)
    ntail = jnp.asarray(_LE_TAIL_NP)
    temps_b = jnp.broadcast_to(temperatures[:, None], (_B, 128))
    out_sc, _, _ = _sc_sample(logits, le, ltail, ntail, temps_b)
    out_tc = _tc_sample(logits, le, temps_b)
    return jnp.concatenate([out_sc[:, 0], out_tc[:, 0]])
